# baseline (speedup 1.0000x reference)
import functools

import jax
import jax.numpy as jnp
from jax import lax
from jax.experimental import pallas as pl
from jax.experimental.pallas import tpu as pltpu
from jax.experimental.pallas import tpu_sc as plsc

H = 128
NC = 2
NS = 16
NW = NC * NS
C = 128


def _emb_body(nchunk, token_hbm, comb_hbm, tidx_hbm, cidx_hbm, out_hbm,
              tix_all, cix_all, bufs, sems):
    a0, b0, o0, a1, b1, o1 = bufs
    sg0, sg1, st0, st1 = sems
    wid = lax.axis_index("s") * NC + lax.axis_index("c")
    base = wid * (nchunk * C)

    pltpu.sync_copy(tidx_hbm.at[wid], tix_all)
    pltpu.sync_copy(cidx_hbm.at[wid], cix_all)

    def start_gather(g, buf_a, buf_b, sem):
        pltpu.async_copy(token_hbm.at[tix_all.at[g]], buf_a, sem)
        pltpu.async_copy(comb_hbm.at[cix_all.at[g]], buf_b, sem)

    def wait_gather(g, buf_a, buf_b, sem):
        pltpu.make_async_copy(token_hbm.at[tix_all.at[g]], buf_a, sem).wait()
        pltpu.make_async_copy(comb_hbm.at[cix_all.at[g]], buf_b, sem).wait()

    def out_slice(g):
        return out_hbm.at[pl.ds(base + g * C, C)]

    def add_chunk(buf_a, buf_b, buf_o):
        @plsc.parallel_loop(0, C, step=1, unroll=8)
        def _(r):
            for j in range(H // 16):
                sl = pl.ds(j * 16, 16)
                buf_o[r, sl] = buf_a[r, sl] + buf_b[r, sl]

    start_gather(0, a0, b0, sg0)
    start_gather(1, a1, b1, sg1)

    def pair(k, carry):
        g0 = 2 * k
        g1 = g0 + 1

        wait_gather(g0, a0, b0, sg0)
        pltpu.async_copy(a0, out_slice(g0), st0)

        wait_gather(g1, a1, b1, sg1)
        pltpu.async_copy(a1, out_slice(g1), st1)

        @pl.when(k > 0)
        def _():
            pltpu.make_async_copy(o0, out_slice(g0 - 2), st0).wait()
            pltpu.make_async_copy(o1, out_slice(g1 - 2), st1).wait()

        @pl.when(k < nchunk // 2 - 1)
        def _():
            start_gather(g0 + 2, a0, b0, sg0)
            start_gather(g1 + 2, a1, b1, sg1)
        return carry

    lax.fori_loop(0, nchunk // 2, pair, 0, unroll=False)

    pltpu.make_async_copy(o0, out_slice(nchunk - 2), st0).wait()
    pltpu.make_async_copy(o1, out_slice(nchunk - 1), st1).wait()


def kernel(sentences, segments, token_table, segment_table, positional_embedding):
    batch, seq = sentences.shape
    bs = batch * seq
    assert bs % (NW * C) == 0
    nchunk = bs // (NW * C)
    assert nchunk % 2 == 0

    comb = (segment_table[:, None, :] + positional_embedding[0, :seq, :][None]
            ).reshape(2 * seq, H)
    tidx = sentences.reshape(NW, nchunk, C).astype(jnp.int32)
    cidx = (segments * seq + jnp.arange(seq, dtype=segments.dtype)[None, :]
            ).reshape(NW, nchunk, C).astype(jnp.int32)

    mesh = plsc.VectorSubcoreMesh(core_axis_name="c", subcore_axis_name="s")
    run = pl.kernel(
        functools.partial(_emb_body, nchunk),
        out_type=jax.ShapeDtypeStruct((bs, H), jnp.float32),
        mesh=mesh,
        scratch_types=[
            pltpu.VMEM((nchunk, C), jnp.int32),
            pltpu.VMEM((nchunk, C), jnp.int32),
            tuple(pltpu.VMEM((C, H), jnp.float32) for _ in range(6)),
            tuple(pltpu.SemaphoreType.DMA for _ in range(4)),
        ],
    )
    out = run(token_table, comb, tidx, cidx)
    return out.reshape(batch, seq, H)


# speedup vs baseline: 1.2734x; 1.2734x over previous
import functools

import jax
import jax.numpy as jnp
from jax import lax
from jax.experimental import pallas as pl
from jax.experimental.pallas import tpu as pltpu
from jax.experimental.pallas import tpu_sc as plsc

H = 128
NC = 2
NS = 16
NW = NC * NS
C = 80


def _emb_body(nchunk, seq, token_hbm, pos_hbm, delta_hbm, segb_hbm, tidx_hbm,
              out_hbm, tix_all, pos_v, delta_v, bufs, sems):
    a0, o0, sb0, a1, o1, sb1 = bufs
    sg0, sg1, st0, st1 = sems
    wid = lax.axis_index("s") * NC + lax.axis_index("c")
    base = wid * (nchunk * C)

    pltpu.sync_copy(pos_hbm, pos_v)
    pltpu.sync_copy(delta_hbm, delta_v)
    pltpu.sync_copy(tidx_hbm.at[wid], tix_all)

    def start_gather(g, buf_a, sb, sem):
        pltpu.async_copy(token_hbm.at[tix_all.at[g]], buf_a, sem)
        pltpu.async_copy(segb_hbm.at[wid].at[g], sb, sem)

    def wait_gather(g, buf_a, sb, sem):
        pltpu.make_async_copy(token_hbm.at[tix_all.at[g]], buf_a, sem).wait()
        pltpu.make_async_copy(segb_hbm.at[wid].at[g], sb, sem).wait()

    def out_slice(g):
        return out_hbm.at[pl.ds(base + g * C, C)]

    def add_chunk(g, buf_a, sb, buf_o):
        s_off = lax.rem(base + g * C, seq)

        @plsc.parallel_loop(0, C, step=1, unroll=4)
        def _(r):
            seg_splat = sb[r, pl.ds(0, 16)]
            pr = s_off + r
            for j in range(H // 16):
                sl = pl.ds(j * 16, 16)
                buf_o[r, sl] = (buf_a[r, sl] + pos_v[pr, sl]
                                + seg_splat * delta_v[sl])

    start_gather(0, a0, sb0, sg0)
    start_gather(1, a1, sb1, sg1)

    def pair(k, carry):
        g0 = 2 * k
        g1 = g0 + 1

        wait_gather(g0, a0, sb0, sg0)

        @pl.when(k > 0)
        def _():
            pltpu.make_async_copy(o0, out_slice(g0 - 2), st0).wait()

        add_chunk(g0, a0, sb0, o0)

        @pl.when(k < nchunk // 2 - 1)
        def _():
            start_gather(g0 + 2, a0, sb0, sg0)

        pltpu.async_copy(o0, out_slice(g0), st0)

        wait_gather(g1, a1, sb1, sg1)

        @pl.when(k > 0)
        def _():
            pltpu.make_async_copy(o1, out_slice(g1 - 2), st1).wait()

        add_chunk(g1, a1, sb1, o1)

        @pl.when(k < nchunk // 2 - 1)
        def _():
            start_gather(g1 + 2, a1, sb1, sg1)

        pltpu.async_copy(o1, out_slice(g1), st1)
        return carry

    lax.fori_loop(0, nchunk // 2, pair, 0, unroll=False)

    pltpu.make_async_copy(o0, out_slice(nchunk - 2), st0).wait()
    pltpu.make_async_copy(o1, out_slice(nchunk - 1), st1).wait()


def kernel(sentences, segments, token_table, segment_table, positional_embedding):
    batch, seq = sentences.shape
    bs = batch * seq
    assert bs % (NW * C) == 0
    nchunk = bs // (NW * C)
    assert nchunk % 2 == 0

    pos_used = positional_embedding[0, :seq, :]
    pos_ext = (jnp.concatenate([pos_used, pos_used[:C]], axis=0)
               + segment_table[0][None, :])
    delta = segment_table[1] - segment_table[0]
    segb = jnp.broadcast_to(
        segments.reshape(NW, nchunk, C, 1).astype(jnp.float32),
        (NW, nchunk, C, 16))
    tidx = sentences.reshape(NW, nchunk, C).astype(jnp.int32)

    mesh = plsc.VectorSubcoreMesh(core_axis_name="c", subcore_axis_name="s")
    run = pl.kernel(
        functools.partial(_emb_body, nchunk, seq),
        out_type=jax.ShapeDtypeStruct((bs, H), jnp.float32),
        mesh=mesh,
        scratch_types=[
            pltpu.VMEM((nchunk, C), jnp.int32),
            pltpu.VMEM((seq + C, H), jnp.float32),
            pltpu.VMEM((H,), jnp.float32),
            tuple([pltpu.VMEM((C, H), jnp.float32),
                   pltpu.VMEM((C, H), jnp.float32),
                   pltpu.VMEM((C, 16), jnp.float32)] * 2),
            tuple(pltpu.SemaphoreType.DMA for _ in range(4)),
        ],
    )
    out = run(token_table, pos_ext, delta, segb, tidx)
    return out.reshape(batch, seq, H)
